# Initial kernel scaffold; baseline (speedup 1.0000x reference)
import jax
import jax.numpy as jnp
from jax.experimental import pallas as pl


def kernel(indices, masks, num_classes, ignore_index):
    raise NotImplementedError("write your pallas kernel here")


# speedup vs baseline: 4.0889x; 4.0889x over previous
import functools

import jax
import jax.numpy as jnp
from jax import lax
from jax.experimental import pallas as pl
from jax.experimental.pallas import tpu as pltpu
from jax.experimental.pallas import tpu_sc as plsc

K = 8192
C = 21
R = 24
NBINS = R * K
NTOPK = 64
EPS = 1e-6
NTOK = 64 * 32 * 32
NCORES = 2
NSUB = 16
NW = NCORES * NSUB
TPW = NTOK // NW
NCHUNK = TPW // 16
ZPW = NBINS // NSUB


def _sc_hist_body(idx_hbm, msk_hbm, zeros_hbm, out_hbm,
                  idx_v, msk_v, bins_v, ones_v, table_sh):
    c = lax.axis_index("c")
    s = lax.axis_index("s")
    wid = c * NSUB + s
    base = wid * TPW
    for i in range(8):
        ones_v[pl.ds(i * 16, 16)] = jnp.ones((16,), jnp.float32)
    pltpu.sync_copy(zeros_hbm.at[pl.ds(s * ZPW, ZPW)],
                    table_sh.at[pl.ds(s * ZPW, ZPW)])
    pltpu.sync_copy(idx_hbm.at[pl.ds(base, TPW)], idx_v)
    pltpu.sync_copy(msk_hbm.at[pl.ds(base, TPW)], msk_v)
    for t in range(NCHUNK):
        mi = msk_v[pl.ds(t * 16, 16)]
        ii = idx_v[pl.ds(t * 16, 16)]
        cls = jnp.minimum(mi, C)
        bins_v[t // 8, pl.ds((t % 8) * 16, 16)] = cls * K + ii
    plsc.subcore_barrier()
    for j in range(16):
        pltpu.sync_copy(ones_v, table_sh.at[bins_v.at[j]], add=True)
    plsc.subcore_barrier()
    pltpu.sync_copy(table_sh.at[pl.ds(s * ZPW, ZPW)],
                    out_hbm.at[c, pl.ds(s * ZPW, ZPW)])


@functools.cache
def _make_sc_hist():
    return pl.kernel(
        _sc_hist_body,
        out_type=jax.ShapeDtypeStruct((NCORES, NBINS), jnp.float32),
        mesh=plsc.VectorSubcoreMesh(core_axis_name="c", subcore_axis_name="s",
                                    num_cores=NCORES, num_subcores=NSUB),
        scratch_types=[
            pltpu.VMEM((TPW,), jnp.int32),
            pltpu.VMEM((TPW,), jnp.int32),
            pltpu.VMEM((16, NCHUNK), jnp.int32),
            pltpu.VMEM((NCHUNK,), jnp.float32),
            pltpu.VMEM_SHARED((NBINS,), jnp.float32),
        ],
    )


def _tc_body(p0_ref, p1_ref, score_ref, ids_ref, val_ref):
    j = p0_ref[...] + p1_ref[...]
    rows = lax.broadcasted_iota(jnp.int32, (R, K), 0)
    cols = lax.broadcasted_iota(jnp.int32, (R, K), 1)
    jm = jnp.where(rows < C, j, 0.0)
    total = jnp.sum(jm, axis=0, keepdims=True)
    ctx = total - j
    tgt_tot = jnp.maximum(jnp.sum(j, axis=1, keepdims=True), 1.0)
    ctx_tot = jnp.maximum(jnp.sum(ctx, axis=1, keepdims=True), 1.0)
    score = jnp.log((j / tgt_tot + EPS) / (ctx / ctx_tot + EPS))
    neg_inf = jnp.float32(-jnp.inf)
    score = jnp.where(j >= 1.0, score, neg_inf)
    score_ref[...] = score
    alive = rows >= 0
    kcols = lax.broadcasted_iota(jnp.int32, (R, NTOPK), 1)
    ids_acc = jnp.zeros((R, NTOPK), jnp.int32)
    val_acc = jnp.zeros((R, NTOPK), jnp.int32)
    for step in range(NTOPK):
        masked = jnp.where(alive, score, neg_inf)
        m = jnp.max(masked, axis=1, keepdims=True)
        is_m = alive & (masked == m)
        jmin = jnp.min(jnp.where(is_m, cols, K), axis=1, keepdims=True)
        ids_acc = jnp.where(kcols == step, jmin, ids_acc)
        val_acc = jnp.where(kcols == step,
                            (m > neg_inf).astype(jnp.int32), val_acc)
        alive = alive & (cols != jmin)
    ids_ref[...] = ids_acc
    val_ref[...] = val_acc


_tc_select = pl.pallas_call(
    _tc_body,
    out_shape=(
        jax.ShapeDtypeStruct((R, K), jnp.float32),
        jax.ShapeDtypeStruct((R, NTOPK), jnp.int32),
        jax.ShapeDtypeStruct((R, NTOPK), jnp.int32),
    ),
)


def kernel(indices, masks, num_classes, ignore_index):
    flat_idx = indices.reshape(-1).astype(jnp.int32)
    flat_msk = masks.reshape(-1).astype(jnp.int32)
    zeros = jnp.zeros((NBINS,), jnp.float32)
    parts = _make_sc_hist()(flat_idx, flat_msk, zeros)
    p = parts.reshape(NCORES, R, K)
    score, ids, val = _tc_select(p[0], p[1])
    return ids[:C], val[:C] != 0, score[:C]


# speedup vs baseline: 4.9213x; 1.2036x over previous
import functools

import jax
import jax.numpy as jnp
from jax import lax
from jax.experimental import pallas as pl
from jax.experimental.pallas import tpu as pltpu
from jax.experimental.pallas import tpu_sc as plsc

K = 8192
C = 21
R = 24
NBINS = R * K
NTOPK = 64
EPS = 1e-6
NTOK = 64 * 32 * 32
NCORES = 2
NSUB = 16
NW = NCORES * NSUB
TPW = NTOK // NW
NCHUNK = TPW // 16
ZPW = NBINS // NSUB


def _sc_hist_body(idx_hbm, msk_hbm, zeros_hbm, out_hbm,
                  idx_v, msk_v, bins_v, ones_v, table_sh):
    c = lax.axis_index("c")
    s = lax.axis_index("s")
    wid = c * NSUB + s
    base = wid * TPW
    for i in range(8):
        ones_v[pl.ds(i * 16, 16)] = jnp.ones((16,), jnp.float32)
    pltpu.sync_copy(zeros_hbm.at[pl.ds(s * ZPW, ZPW)],
                    table_sh.at[pl.ds(s * ZPW, ZPW)])
    pltpu.sync_copy(idx_hbm.at[pl.ds(base, TPW)], idx_v)
    pltpu.sync_copy(msk_hbm.at[pl.ds(base, TPW)], msk_v)
    for t in range(NCHUNK):
        mi = msk_v[pl.ds(t * 16, 16)]
        ii = idx_v[pl.ds(t * 16, 16)]
        cls = jnp.minimum(mi, C)
        bins_v[t // 8, pl.ds((t % 8) * 16, 16)] = cls * K + ii
    plsc.subcore_barrier()
    for j in range(16):
        pltpu.sync_copy(ones_v, table_sh.at[bins_v.at[j]], add=True)
    plsc.subcore_barrier()
    pltpu.sync_copy(table_sh.at[pl.ds(s * ZPW, ZPW)],
                    out_hbm.at[c, pl.ds(s * ZPW, ZPW)])


@functools.cache
def _make_sc_hist():
    return pl.kernel(
        _sc_hist_body,
        out_type=jax.ShapeDtypeStruct((NCORES, NBINS), jnp.float32),
        mesh=plsc.VectorSubcoreMesh(core_axis_name="c", subcore_axis_name="s",
                                    num_cores=NCORES, num_subcores=NSUB),
        scratch_types=[
            pltpu.VMEM((TPW,), jnp.int32),
            pltpu.VMEM((TPW,), jnp.int32),
            pltpu.VMEM((16, NCHUNK), jnp.int32),
            pltpu.VMEM((NCHUNK,), jnp.float32),
            pltpu.VMEM_SHARED((NBINS,), jnp.float32),
        ],
    )


def _tc_body(p_ref, score_ref, ids_ref, val_ref):
    j = p_ref[:R] + p_ref[R:]
    rows = lax.broadcasted_iota(jnp.int32, (R, K), 0)
    cols = lax.broadcasted_iota(jnp.int32, (R, K), 1)
    jm = jnp.where(rows < C, j, 0.0)
    total = jnp.sum(jm, axis=0, keepdims=True)
    ctx = total - j
    tgt_tot = jnp.maximum(jnp.sum(j, axis=1, keepdims=True), 1.0)
    ctx_tot = jnp.maximum(jnp.sum(ctx, axis=1, keepdims=True), 1.0)
    score = jnp.log((j / tgt_tot + EPS) / (ctx / ctx_tot + EPS))
    neg_inf = jnp.float32(-jnp.inf)
    score_ref[...] = jnp.where(j[:C] >= 1.0, score[:C], neg_inf)
    work = jnp.where(j >= 1.0, score, -(10000.0 + cols.astype(jnp.float32)))
    kcols = lax.broadcasted_iota(jnp.int32, (R, NTOPK), 1)
    ids_acc = jnp.zeros((R, NTOPK), jnp.int32)
    val_acc = jnp.zeros((R, NTOPK), jnp.int32)
    for step in range(NTOPK):
        m = jnp.max(work, axis=1, keepdims=True)
        jmin = jnp.min(jnp.where(work == m, cols, K), axis=1, keepdims=True)
        ids_acc = jnp.where(kcols == step, jmin, ids_acc)
        val_acc = jnp.where(kcols == step,
                            (m > -9999.0).astype(jnp.int32), val_acc)
        work = jnp.where(cols == jmin, neg_inf, work)
    ids_ref[...] = ids_acc[:C]
    val_ref[...] = val_acc[:C]


_tc_select = pl.pallas_call(
    _tc_body,
    out_shape=(
        jax.ShapeDtypeStruct((C, K), jnp.float32),
        jax.ShapeDtypeStruct((C, NTOPK), jnp.int32),
        jax.ShapeDtypeStruct((C, NTOPK), jnp.int32),
    ),
)


def kernel(indices, masks, num_classes, ignore_index):
    flat_idx = indices.reshape(-1).astype(jnp.int32)
    flat_msk = masks.reshape(-1).astype(jnp.int32)
    zeros = jnp.zeros((NBINS,), jnp.float32)
    parts = _make_sc_hist()(flat_idx, flat_msk, zeros)
    score, ids, val = _tc_select(parts.reshape(NCORES * R, K))
    return ids, val != 0, score


# speedup vs baseline: 5.0438x; 1.0249x over previous
import functools

import jax
import jax.numpy as jnp
from jax import lax
from jax.experimental import pallas as pl
from jax.experimental.pallas import tpu as pltpu
from jax.experimental.pallas import tpu_sc as plsc

K = 8192
C = 21
R = 24
NBINS = R * K
NTOPK = 64
EPS = 1e-6
NTOK = 64 * 32 * 32
NCORES = 2
NSUB = 16
NW = NCORES * NSUB
TPW = NTOK // NW
NCHUNK = TPW // 16
ZPW = NBINS // NSUB


def _sc_hist_body(idx_hbm, msk_hbm, out_hbm,
                  idx_v, msk_v, bins_v, ones_v, zeros_v, table_sh):
    c = lax.axis_index("c")
    s = lax.axis_index("s")
    wid = c * NSUB + s
    base = wid * TPW
    for i in range(8):
        ones_v[pl.ds(i * 16, 16)] = jnp.ones((16,), jnp.float32)
    for i in range(TPW // 16):
        zeros_v[pl.ds(i * 16, 16)] = jnp.zeros((16,), jnp.float32)
    for i in range(ZPW // TPW):
        pltpu.sync_copy(zeros_v,
                        table_sh.at[pl.ds(s * ZPW + i * TPW, TPW)])
    pltpu.sync_copy(idx_hbm.at[pl.ds(base, TPW)], idx_v)
    pltpu.sync_copy(msk_hbm.at[pl.ds(base, TPW)], msk_v)
    for t in range(NCHUNK):
        mi = msk_v[pl.ds(t * 16, 16)]
        ii = idx_v[pl.ds(t * 16, 16)]
        cls = jnp.minimum(mi, C)
        bins_v[t // 8, pl.ds((t % 8) * 16, 16)] = cls * K + ii
    plsc.subcore_barrier()
    for j in range(16):
        pltpu.sync_copy(ones_v, table_sh.at[bins_v.at[j]], add=True)
    plsc.subcore_barrier()
    pltpu.sync_copy(table_sh.at[pl.ds(s * ZPW, ZPW)],
                    out_hbm.at[c, pl.ds(s * ZPW, ZPW)])


@functools.cache
def _make_sc_hist():
    return pl.kernel(
        _sc_hist_body,
        out_type=jax.ShapeDtypeStruct((NCORES, NBINS), jnp.float32),
        mesh=plsc.VectorSubcoreMesh(core_axis_name="c", subcore_axis_name="s",
                                    num_cores=NCORES, num_subcores=NSUB),
        scratch_types=[
            pltpu.VMEM((TPW,), jnp.int32),
            pltpu.VMEM((TPW,), jnp.int32),
            pltpu.VMEM((16, NCHUNK), jnp.int32),
            pltpu.VMEM((NCHUNK,), jnp.float32),
            pltpu.VMEM((TPW,), jnp.float32),
            pltpu.VMEM_SHARED((NBINS,), jnp.float32),
        ],
    )


def _tc_body(p_ref, score_ref, ids_ref, val_ref):
    j = p_ref[:R] + p_ref[R:]
    rows = lax.broadcasted_iota(jnp.int32, (R, K), 0)
    cols = lax.broadcasted_iota(jnp.int32, (R, K), 1)
    jm = jnp.where(rows < C, j, 0.0)
    total = jnp.sum(jm, axis=0, keepdims=True)
    ctx = total - j
    tgt_tot = jnp.maximum(jnp.sum(j, axis=1, keepdims=True), 1.0)
    ctx_tot = jnp.maximum(jnp.sum(ctx, axis=1, keepdims=True), 1.0)
    score = jnp.log((j / tgt_tot + EPS) / (ctx / ctx_tot + EPS))
    neg_inf = jnp.float32(-jnp.inf)
    score_ref[...] = jnp.where(j[:C] >= 1.0, score[:C], neg_inf)
    work = jnp.where(j >= 1.0, score, -(10000.0 + cols.astype(jnp.float32)))
    kcols = lax.broadcasted_iota(jnp.int32, (R, NTOPK), 1)
    ids_acc = jnp.zeros((R, NTOPK), jnp.int32)
    val_acc = jnp.zeros((R, NTOPK), jnp.int32)
    jmin = jnp.full((R, 1), -1, jnp.int32)
    for step in range(NTOPK):
        work = jnp.where(cols == jmin, neg_inf, work)
        m = jnp.max(work, axis=1, keepdims=True)
        jmin = jnp.min(jnp.where(work == m, cols, K), axis=1, keepdims=True)
        ids_acc = jnp.where(kcols == step, jmin, ids_acc)
        val_acc = jnp.where(kcols == step,
                            (m > -9999.0).astype(jnp.int32), val_acc)
    ids_ref[...] = ids_acc[:C]
    val_ref[...] = val_acc[:C] != 0


_tc_select = pl.pallas_call(
    _tc_body,
    out_shape=(
        jax.ShapeDtypeStruct((C, K), jnp.float32),
        jax.ShapeDtypeStruct((C, NTOPK), jnp.int32),
        jax.ShapeDtypeStruct((C, NTOPK), jnp.bool_),
    ),
)


def kernel(indices, masks, num_classes, ignore_index):
    flat_idx = indices.reshape(-1).astype(jnp.int32)
    flat_msk = masks.reshape(-1).astype(jnp.int32)
    parts = _make_sc_hist()(flat_idx, flat_msk)
    score, ids, val = _tc_select(parts.reshape(NCORES * R, K))
    return ids, val, score


# speedup vs baseline: 5.2598x; 1.0428x over previous
import functools

import jax
import jax.numpy as jnp
from jax import lax
from jax.experimental import pallas as pl
from jax.experimental.pallas import tpu as pltpu
from jax.experimental.pallas import tpu_sc as plsc

K = 8192
C = 21
R = 24
NBINS = R * K
NTOPK = 64
EPS = 1e-6
NTOK = 64 * 32 * 32
NCORES = 2
NSUB = 16
NW = NCORES * NSUB
TPW = NTOK // NW
NCHUNK = TPW // 16
ZPW = NBINS // NSUB


def _sc_hist_body(idx_hbm, msk_hbm, out_hbm,
                  idx_v, msk_v, bins_v, ones_v, zeros_v, table_sh,
                  sem_ld, sem_z, sem_sc):
    c = lax.axis_index("c")
    s = lax.axis_index("s")
    wid = c * NSUB + s
    base = wid * TPW
    ld_idx = pltpu.async_copy(idx_hbm.at[pl.ds(base, TPW)], idx_v, sem_ld)
    ld_msk = pltpu.async_copy(msk_hbm.at[pl.ds(base, TPW)], msk_v, sem_ld)
    for i in range(8):
        ones_v[pl.ds(i * 16, 16)] = jnp.ones((16,), jnp.float32)
    for i in range(TPW // 16):
        zeros_v[pl.ds(i * 16, 16)] = jnp.zeros((16,), jnp.float32)
    zcopies = [
        pltpu.async_copy(zeros_v, table_sh.at[pl.ds(s * ZPW + i * TPW, TPW)],
                         sem_z)
        for i in range(ZPW // TPW)
    ]
    ld_idx.wait()
    ld_msk.wait()
    for t in range(NCHUNK):
        mi = msk_v[pl.ds(t * 16, 16)]
        ii = idx_v[pl.ds(t * 16, 16)]
        cls = jnp.minimum(mi, C)
        bins_v[t // 8, pl.ds((t % 8) * 16, 16)] = cls * K + ii
    for cp in zcopies:
        cp.wait()
    plsc.subcore_barrier()
    scopies = [
        pltpu.async_copy(ones_v, table_sh.at[bins_v.at[j]], sem_sc, add=True)
        for j in range(16)
    ]
    for cp in scopies:
        cp.wait()
    plsc.subcore_barrier()
    pltpu.sync_copy(table_sh.at[pl.ds(s * ZPW, ZPW)],
                    out_hbm.at[c, pl.ds(s * ZPW, ZPW)])


@functools.cache
def _make_sc_hist():
    return pl.kernel(
        _sc_hist_body,
        out_type=jax.ShapeDtypeStruct((NCORES, NBINS), jnp.float32),
        mesh=plsc.VectorSubcoreMesh(core_axis_name="c", subcore_axis_name="s",
                                    num_cores=NCORES, num_subcores=NSUB),
        scratch_types=[
            pltpu.VMEM((TPW,), jnp.int32),
            pltpu.VMEM((TPW,), jnp.int32),
            pltpu.VMEM((16, NCHUNK), jnp.int32),
            pltpu.VMEM((NCHUNK,), jnp.float32),
            pltpu.VMEM((TPW,), jnp.float32),
            pltpu.VMEM_SHARED((NBINS,), jnp.float32),
            pltpu.SemaphoreType.DMA,
            pltpu.SemaphoreType.DMA,
            pltpu.SemaphoreType.DMA,
        ],
    )


def _tc_body(p_ref, score_ref, ids_ref, val_ref):
    j = p_ref[:R] + p_ref[R:]
    rows = lax.broadcasted_iota(jnp.int32, (R, K), 0)
    cols = lax.broadcasted_iota(jnp.int32, (R, K), 1)
    jm = jnp.where(rows < C, j, 0.0)
    total = jnp.sum(jm, axis=0, keepdims=True)
    ctx = total - j
    tgt_tot = jnp.maximum(jnp.sum(j, axis=1, keepdims=True), 1.0)
    ctx_tot = jnp.maximum(jnp.sum(ctx, axis=1, keepdims=True), 1.0)
    score = jnp.log((j / tgt_tot + EPS) / (ctx / ctx_tot + EPS))
    neg_inf = jnp.float32(-jnp.inf)
    score_ref[...] = jnp.where(j[:C] >= 1.0, score[:C], neg_inf)
    work = jnp.where(j >= 1.0, score, -(10000.0 + cols.astype(jnp.float32)))
    kcols = lax.broadcasted_iota(jnp.int32, (R, NTOPK), 1)
    ids_acc = jnp.zeros((R, NTOPK), jnp.int32)
    val_acc = jnp.zeros((R, NTOPK), jnp.int32)
    jmin = jnp.full((R, 1), -1, jnp.int32)
    for step in range(NTOPK):
        work = jnp.where(cols == jmin, neg_inf, work)
        m = jnp.max(work, axis=1, keepdims=True)
        jmin = jnp.min(jnp.where(work == m, cols, K), axis=1, keepdims=True)
        ids_acc = jnp.where(kcols == step, jmin, ids_acc)
        val_acc = jnp.where(kcols == step,
                            (m > -9999.0).astype(jnp.int32), val_acc)
    ids_ref[...] = ids_acc[:C]
    val_ref[...] = val_acc[:C] != 0


_tc_select = pl.pallas_call(
    _tc_body,
    out_shape=(
        jax.ShapeDtypeStruct((C, K), jnp.float32),
        jax.ShapeDtypeStruct((C, NTOPK), jnp.int32),
        jax.ShapeDtypeStruct((C, NTOPK), jnp.bool_),
    ),
)


def kernel(indices, masks, num_classes, ignore_index):
    flat_idx = indices.reshape(-1).astype(jnp.int32)
    flat_msk = masks.reshape(-1).astype(jnp.int32)
    parts = _make_sc_hist()(flat_idx, flat_msk)
    score, ids, val = _tc_select(parts.reshape(NCORES * R, K))
    return ids, val, score
